# baseline (speedup 1.0000x reference)
import functools
import jax
import jax.numpy as jnp
from jax import lax
from jax.experimental import pallas as pl
from jax.experimental.pallas import tpu as pltpu
from jax.experimental.pallas import tpu_sc as plsc

B, M, E, R, D = 4, 2500, 160000, 512, 256
BPAD = 2560
PAD_ROWS = B * BPAD
SLAB_E = 80
SLAB_CELLS = SLAB_E * R
NSLAB = PAD_ROWS // SLAB_E
NW = 32
CAP = 256
CELLS_PER_TILE = 2 * E // NW
FACTS_PER_TILE = E // NW


_GATHER_DNUMS = lax.GatherDimensionNumbers(
    offset_dims=(), collapsed_slice_dims=(0,), start_index_map=(0,))


def _take16(v, idx):
    return lax.gather(v, idx[:, None], _GATHER_DNUMS, slice_sizes=(1,),
                      mode=lax.GatherScatterMode.PROMISE_IN_BOUNDS)


def _prep_body(h_ref, t_ref, i_ref, r_ref, hc_ref, tc_ref):
    common = r_ref[...] + i_ref[...] * (BPAD * R)
    hc_ref[...] = h_ref[...] * R + common
    tc_ref[...] = t_ref[...] * R + common


_prep_call = pl.pallas_call(
    _prep_body,
    in_specs=[
        pl.BlockSpec((E,), lambda: (0,)),
        pl.BlockSpec((E,), lambda: (0,)),
        pl.BlockSpec((E,), lambda: (0,)),
        pl.BlockSpec((E,), lambda: (0,)),
    ],
    out_specs=[
        pl.BlockSpec((E,), lambda: (0,)),
        pl.BlockSpec((E,), lambda: (0,)),
    ],
    out_shape=[
        jax.ShapeDtypeStruct((E,), jnp.int32),
        jax.ShapeDtypeStruct((E,), jnp.int32),
    ],
)


_sc_mesh = plsc.VectorSubcoreMesh(core_axis_name="c", subcore_axis_name="s")


@functools.partial(
    pl.kernel,
    out_type=(
        jax.ShapeDtypeStruct((NW * NSLAB, CAP), jnp.int32),
        jax.ShapeDtypeStruct((NW * NSLAB,), jnp.int32),
        jax.ShapeDtypeStruct((NW, R), jnp.float32),
    ),
    mesh=_sc_mesh,
    compiler_params=pltpu.CompilerParams(needs_layout_passes=False),
    scratch_types=(
        pltpu.VMEM((NSLAB, CAP), jnp.int32),
        pltpu.VMEM((CELLS_PER_TILE,), jnp.int32),
        pltpu.VMEM((NSLAB,), jnp.int32),
        pltpu.VMEM((R,), jnp.float32),
    ),
)
def _sc_bin(hcells_hbm, tcells_hbm, rels_hbm, staged_out, lens_out, cnt_out,
            binbuf, cellbuf, cursors, cnthist):
    wid = lax.axis_index("s") * 2 + lax.axis_index("c")
    ones = jnp.ones((16,), jnp.float32)
    iota = lax.iota(jnp.int32, 16)

    fbase = wid * FACTS_PER_TILE
    pltpu.sync_copy(hcells_hbm.at[pl.ds(fbase, FACTS_PER_TILE)],
                    cellbuf.at[pl.ds(0, FACTS_PER_TILE)])
    pltpu.sync_copy(tcells_hbm.at[pl.ds(fbase, FACTS_PER_TILE)],
                    cellbuf.at[pl.ds(FACTS_PER_TILE, FACTS_PER_TILE)])

    for k in range(NSLAB // 16):
        cursors[pl.ds(k * 16, 16)] = jnp.zeros((16,), jnp.int32)

    def bin_body(i, _):
        c = cellbuf[pl.ds(i * 16, 16)]
        row = lax.shift_right_logical(c, 9)
        b = lax.shift_right_logical(row * 13108, 20)
        sk, sv = plsc.sort_key_val(b, c)
        prev = _take16(sk, jnp.maximum(iota - 1, 0))
        boundary = (iota == 0) | (sk != prev)
        firstpos = plsc.cummax(jnp.where(boundary, iota, 0))
        rank = iota - firstpos
        base = plsc.load_gather(cursors, [sk])
        pos = jnp.minimum(base + rank, CAP - 1)
        plsc.store_scatter(binbuf, [sk, pos], sv)
        nxt = _take16(sk, jnp.minimum(iota + 1, 15))
        run_end = (iota == 15) | (sk != nxt)
        plsc.store_scatter(cursors, [sk], pos + 1, mask=run_end)
        return 0
    lax.fori_loop(0, CELLS_PER_TILE // 16, bin_body, 0)

    pltpu.sync_copy(binbuf, staged_out.at[pl.ds(wid * NSLAB, NSLAB)])
    pltpu.sync_copy(cursors, lens_out.at[pl.ds(wid * NSLAB, NSLAB)])

    @plsc.parallel_loop(0, R, step=16, unroll=8)
    def _(i):
        cnthist[pl.ds(i, 16)] = jnp.zeros((16,), jnp.float32)

    pltpu.sync_copy(rels_hbm.at[pl.ds(fbase, FACTS_PER_TILE)],
                    cellbuf.at[pl.ds(0, FACTS_PER_TILE)])

    @plsc.parallel_loop(0, FACTS_PER_TILE, step=16, unroll=8)
    def _(i):
        r = cellbuf[pl.ds(i, 16)]
        m = iota < (FACTS_PER_TILE - i)
        plsc.addupdate_scatter(cnthist, [jnp.where(m, r, 0)], ones, mask=m)

    pltpu.sync_copy(cnthist, cnt_out.at[wid])


@functools.partial(
    pl.kernel,
    out_type=jax.ShapeDtypeStruct((PAD_ROWS, R), jnp.float32),
    mesh=_sc_mesh,
    compiler_params=pltpu.CompilerParams(needs_layout_passes=False),
    scratch_types=(
        pltpu.VMEM((SLAB_E, R), jnp.float32),
        pltpu.VMEM((SLAB_E, R), jnp.float32),
        pltpu.VMEM((NW * CAP,), jnp.int32),
        pltpu.VMEM((NW * CAP,), jnp.int32),
        pltpu.VMEM((NW * NSLAB,), jnp.int32),
        pltpu.SemaphoreType.DMA,
        pltpu.SemaphoreType.DMA,
        pltpu.SemaphoreType.DMA,
        pltpu.SemaphoreType.DMA,
    ),
)
def _sc_hist(staged_hbm, lens_hbm, zslab_hbm, c_out,
             hist0, hist1, segA, segB, lensbuf, semA, semB, semf0, semf1):
    wid = lax.axis_index("s") * 2 + lax.axis_index("c")
    ones = jnp.ones((16,), jnp.float32)
    iota = lax.iota(jnp.int32, 16)

    hists = (hist0, hist1)
    segs = (segA, segB)
    seg_sems = (semA, semB)
    flush_sems = (semf0, semf1)

    def fire_segs(k, buf_idx):
        sb = wid * 4 + k
        for s in range(NW):
            pltpu.async_copy(staged_hbm.at[s * NSLAB + sb],
                             segs[buf_idx].at[pl.ds(s * CAP, CAP)],
                             seg_sems[buf_idx])

    def drain_segs(k, buf_idx):
        sb = wid * 4 + k
        for s in range(NW):
            pltpu.make_async_copy(staged_hbm.at[s * NSLAB + sb],
                                  segs[buf_idx].at[pl.ds(s * CAP, CAP)],
                                  seg_sems[buf_idx]).wait()

    def process_bin(k, buf_idx):
        seg = segs[buf_idx]
        hist = hists[buf_idx]
        sb = wid * 4 + k
        base = sb * SLAB_CELLS
        for s in range(NW):
            ls = plsc.load_gather(
                lensbuf, [jnp.full((16,), s * NSLAB + sb, jnp.int32)])

            @plsc.parallel_loop(0, CAP, step=16, unroll=8)
            def _(i):
                m = (iota + i) < ls
                c = seg[pl.ds(s * CAP + i, 16)]
                l = c - base
                lm = jnp.where(m, l, 0)
                row = lax.shift_right_logical(lm, 9)
                col = lm & (R - 1)
                plsc.addupdate_scatter(hist, [row, col], ones, mask=m)

    def fire_flush(k, buf_idx):
        sb = wid * 4 + k
        pltpu.async_copy(hists[buf_idx],
                         c_out.at[pl.ds(sb * SLAB_E, SLAB_E)],
                         flush_sems[buf_idx])

    def wait_flush(k, buf_idx):
        sb = wid * 4 + k
        pltpu.make_async_copy(hists[buf_idx],
                              c_out.at[pl.ds(sb * SLAB_E, SLAB_E)],
                              flush_sems[buf_idx]).wait()

    pltpu.sync_copy(lens_hbm, lensbuf)
    fire_segs(0, 0)
    pltpu.sync_copy(zslab_hbm, hist0)
    fire_segs(1, 1)
    pltpu.sync_copy(zslab_hbm, hist1)

    drain_segs(0, 0)
    process_bin(0, 0)
    fire_flush(0, 0)
    fire_segs(2, 0)

    drain_segs(1, 1)
    process_bin(1, 1)
    fire_flush(1, 1)
    fire_segs(3, 1)

    wait_flush(0, 0)
    pltpu.sync_copy(zslab_hbm, hist0)
    drain_segs(2, 0)
    process_bin(2, 0)
    fire_flush(2, 0)

    wait_flush(1, 1)
    pltpu.sync_copy(zslab_hbm, hist1)
    drain_segs(3, 1)
    process_bin(3, 1)
    fire_flush(3, 1)

    wait_flush(2, 0)
    wait_flush(3, 1)


def _tc_body(c_ref, cntp_ref, relf_ref, wkb_ref, bkb_ref, wa_ref, out_ref, v_scr):
    @pl.when(pl.program_id(0) == 0)
    def _():
        rel_val = lax.dot_general(
            relf_ref[...], wkb_ref[...], (((1,), (1,)), ((), ())),
            preferred_element_type=jnp.float32) + bkb_ref[0:1, :]
        attn_in = jnp.concatenate([rel_val, relf_ref[...]], axis=1)
        s8 = lax.dot_general(
            attn_in, wa_ref[...], (((1,), (1,)), ((), ())),
            preferred_element_type=jnp.float32)
        s = s8[:, 0:1]
        s = jnp.where(s >= 0.0, s, 0.2 * s)
        es = jnp.exp(s)
        cnt = jnp.sum(cntp_ref[...], axis=0, keepdims=True)
        denom = lax.dot_general(
            cnt, es, (((1,), (0,)), ((), ())),
            preferred_element_type=jnp.float32) + 1e-9
        v_scr[...] = rel_val * (es / denom)

    x = lax.dot_general(
        c_ref[0:M, :], v_scr[...], (((1,), (0,)), ((), ())),
        preferred_element_type=jnp.float32)
    x = jnp.where(x > 0.0, x, jnp.exp(jnp.minimum(x, 0.0)) - 1.0)
    out_ref[...] = x[None]


_tc_call = pl.pallas_call(
    _tc_body,
    grid=(B,),
    in_specs=[
        pl.BlockSpec((BPAD, R), lambda b: (b, 0)),
        pl.BlockSpec((NW, R), lambda b: (0, 0)),
        pl.BlockSpec((R, D), lambda b: (0, 0)),
        pl.BlockSpec((D, D), lambda b: (0, 0)),
        pl.BlockSpec((8, D), lambda b: (0, 0)),
        pl.BlockSpec((8, 2 * D), lambda b: (0, 0)),
    ],
    out_specs=pl.BlockSpec((1, M, D), lambda b: (b, 0, 0)),
    out_shape=jax.ShapeDtypeStruct((B, M, D), jnp.float32),
    scratch_shapes=[pltpu.VMEM((R, D), jnp.float32)],
)


def kernel(local_entity, batch_heads, batch_rels, batch_tails, batch_ids,
           fact_ids, weight_list, weight_rel_list, rel_features, W_kb, b_kb,
           W_attn):
    heads = batch_heads.astype(jnp.int32)
    tails = batch_tails.astype(jnp.int32)
    ids = batch_ids.astype(jnp.int32)
    rels = batch_rels.astype(jnp.int32)

    hcells, tcells = _prep_call(heads, tails, ids, rels)
    staged, lens, cnt_part = _sc_bin(hcells, tcells, rels)
    zslab = jnp.zeros((SLAB_E, R), jnp.float32)
    c_mat = _sc_hist(staged, lens, zslab)

    bkb8 = jnp.zeros((8, D), jnp.float32).at[0].set(b_kb.astype(jnp.float32))
    wa8 = jnp.zeros((8, 2 * D), jnp.float32).at[0].set(
        W_attn.astype(jnp.float32).reshape(2 * D))

    return _tc_call(c_mat, cnt_part, rel_features, W_kb, bkb8, wa8)


# speedup vs baseline: 1.0625x; 1.0625x over previous
import functools
import jax
import jax.numpy as jnp
from jax import lax
from jax.experimental import pallas as pl
from jax.experimental.pallas import tpu as pltpu
from jax.experimental.pallas import tpu_sc as plsc

B, M, E, R, D = 4, 2500, 160000, 512, 256
BPAD = 2560
PAD_ROWS = B * BPAD
SLAB_E = 160
SLAB_CELLS = SLAB_E * R
NSLAB = PAD_ROWS // SLAB_E
NW = 32
CAP = 512
CELLS_PER_TILE = 2 * E // NW
FACTS_PER_TILE = E // NW


_GATHER_DNUMS = lax.GatherDimensionNumbers(
    offset_dims=(), collapsed_slice_dims=(0,), start_index_map=(0,))


def _take16(v, idx):
    return lax.gather(v, idx[:, None], _GATHER_DNUMS, slice_sizes=(1,),
                      mode=lax.GatherScatterMode.PROMISE_IN_BOUNDS)


def _prep_body(h_ref, t_ref, i_ref, r_ref, hc_ref, tc_ref):
    common = r_ref[...] + i_ref[...] * (BPAD * R)
    hc_ref[...] = h_ref[...] * R + common
    tc_ref[...] = t_ref[...] * R + common


_prep_call = pl.pallas_call(
    _prep_body,
    in_specs=[
        pl.BlockSpec((E,), lambda: (0,)),
        pl.BlockSpec((E,), lambda: (0,)),
        pl.BlockSpec((E,), lambda: (0,)),
        pl.BlockSpec((E,), lambda: (0,)),
    ],
    out_specs=[
        pl.BlockSpec((E,), lambda: (0,)),
        pl.BlockSpec((E,), lambda: (0,)),
    ],
    out_shape=[
        jax.ShapeDtypeStruct((E,), jnp.int32),
        jax.ShapeDtypeStruct((E,), jnp.int32),
    ],
)


_sc_mesh = plsc.VectorSubcoreMesh(core_axis_name="c", subcore_axis_name="s")


@functools.partial(
    pl.kernel,
    out_type=(
        jax.ShapeDtypeStruct((NW * NSLAB, CAP), jnp.int32),
        jax.ShapeDtypeStruct((NW * NSLAB,), jnp.int32),
        jax.ShapeDtypeStruct((NW, R), jnp.float32),
    ),
    mesh=_sc_mesh,
    compiler_params=pltpu.CompilerParams(needs_layout_passes=False),
    scratch_types=(
        pltpu.VMEM((NSLAB, CAP), jnp.int32),
        pltpu.VMEM((CELLS_PER_TILE,), jnp.int32),
        pltpu.VMEM((NSLAB,), jnp.int32),
        pltpu.VMEM((R,), jnp.float32),
    ),
)
def _sc_bin(hcells_hbm, tcells_hbm, rels_hbm, staged_out, lens_out, cnt_out,
            binbuf, cellbuf, cursors, cnthist):
    wid = lax.axis_index("s") * 2 + lax.axis_index("c")
    ones = jnp.ones((16,), jnp.float32)
    iota = lax.iota(jnp.int32, 16)

    fbase = wid * FACTS_PER_TILE
    pltpu.sync_copy(hcells_hbm.at[pl.ds(fbase, FACTS_PER_TILE)],
                    cellbuf.at[pl.ds(0, FACTS_PER_TILE)])
    pltpu.sync_copy(tcells_hbm.at[pl.ds(fbase, FACTS_PER_TILE)],
                    cellbuf.at[pl.ds(FACTS_PER_TILE, FACTS_PER_TILE)])

    for k in range(NSLAB // 16):
        cursors[pl.ds(k * 16, 16)] = jnp.zeros((16,), jnp.int32)

    def bin_body(i, _):
        c = cellbuf[pl.ds(i * 16, 16)]
        row = lax.shift_right_logical(c, 9)
        b = lax.shift_right_logical(row * 6554, 20)
        sk, sv = plsc.sort_key_val(b, c)
        prev = _take16(sk, jnp.maximum(iota - 1, 0))
        boundary = (iota == 0) | (sk != prev)
        firstpos = plsc.cummax(jnp.where(boundary, iota, 0))
        rank = iota - firstpos
        base = plsc.load_gather(cursors, [sk])
        pos = jnp.minimum(base + rank, CAP - 1)
        plsc.store_scatter(binbuf, [sk, pos], sv)
        nxt = _take16(sk, jnp.minimum(iota + 1, 15))
        run_end = (iota == 15) | (sk != nxt)
        plsc.store_scatter(cursors, [sk], pos + 1, mask=run_end)
        return 0
    lax.fori_loop(0, CELLS_PER_TILE // 16, bin_body, 0)

    pltpu.sync_copy(binbuf, staged_out.at[pl.ds(wid * NSLAB, NSLAB)])
    pltpu.sync_copy(cursors, lens_out.at[pl.ds(wid * NSLAB, NSLAB)])

    @plsc.parallel_loop(0, R, step=16, unroll=8)
    def _(i):
        cnthist[pl.ds(i, 16)] = jnp.zeros((16,), jnp.float32)

    pltpu.sync_copy(rels_hbm.at[pl.ds(fbase, FACTS_PER_TILE)],
                    cellbuf.at[pl.ds(0, FACTS_PER_TILE)])

    @plsc.parallel_loop(0, FACTS_PER_TILE, step=16, unroll=8)
    def _(i):
        r = cellbuf[pl.ds(i, 16)]
        m = iota < (FACTS_PER_TILE - i)
        plsc.addupdate_scatter(cnthist, [jnp.where(m, r, 0)], ones, mask=m)

    pltpu.sync_copy(cnthist, cnt_out.at[wid])


@functools.partial(
    pl.kernel,
    out_type=jax.ShapeDtypeStruct((PAD_ROWS, R), jnp.float32),
    mesh=_sc_mesh,
    compiler_params=pltpu.CompilerParams(needs_layout_passes=False),
    scratch_types=(
        pltpu.VMEM((SLAB_E, R), jnp.float32),
        pltpu.VMEM((NW * CAP,), jnp.int32),
        pltpu.VMEM((NW * CAP,), jnp.int32),
        pltpu.VMEM((NW * NSLAB,), jnp.int32),
        pltpu.SemaphoreType.DMA,
        pltpu.SemaphoreType.DMA,
    ),
)
def _sc_hist(staged_hbm, lens_hbm, zslab_hbm, c_out,
             hist, seg0, seg1, lensbuf, sem0, sem1):
    wid = lax.axis_index("s") * 2 + lax.axis_index("c")
    ones = jnp.ones((16,), jnp.float32)
    iota = lax.iota(jnp.int32, 16)
    sb0 = wid * 2
    sb1 = wid * 2 + 1

    pltpu.sync_copy(lens_hbm, lensbuf)
    for s in range(NW):
        pltpu.async_copy(staged_hbm.at[s * NSLAB + sb0],
                         seg0.at[pl.ds(s * CAP, CAP)], sem0)
    pltpu.sync_copy(zslab_hbm, hist)
    for s in range(NW):
        pltpu.make_async_copy(staged_hbm.at[s * NSLAB + sb0],
                              seg0.at[pl.ds(s * CAP, CAP)], sem0).wait()
    for s in range(NW):
        pltpu.async_copy(staged_hbm.at[s * NSLAB + sb1],
                         seg1.at[pl.ds(s * CAP, CAP)], sem1)

    def process_bin(seg, sb):
        base = sb * SLAB_CELLS
        for s in range(NW):
            ls = plsc.load_gather(
                lensbuf, [jnp.full((16,), s * NSLAB + sb, jnp.int32)])

            @plsc.parallel_loop(0, CAP, step=16, unroll=16)
            def _(i):
                m = (iota + i) < ls
                c = seg[pl.ds(s * CAP + i, 16)]
                l = c - base
                lm = jnp.where(m, l, 0)
                row = lax.shift_right_logical(lm, 9)
                col = lm & (R - 1)
                plsc.addupdate_scatter(hist, [row, col], ones, mask=m)

    process_bin(seg0, sb0)
    pltpu.sync_copy(hist, c_out.at[pl.ds(sb0 * SLAB_E, SLAB_E)])
    pltpu.sync_copy(zslab_hbm, hist)

    for s in range(NW):
        pltpu.make_async_copy(staged_hbm.at[s * NSLAB + sb1],
                              seg1.at[pl.ds(s * CAP, CAP)], sem1).wait()
    process_bin(seg1, sb1)
    pltpu.sync_copy(hist, c_out.at[pl.ds(sb1 * SLAB_E, SLAB_E)])


def _tc_body(c_ref, cntp_ref, relf_ref, wkb_ref, bkb_ref, wa_ref, out_ref, v_scr):
    @pl.when(pl.program_id(0) == 0)
    def _():
        rel_val = lax.dot_general(
            relf_ref[...], wkb_ref[...], (((1,), (1,)), ((), ())),
            preferred_element_type=jnp.float32) + bkb_ref[0:1, :]
        attn_in = jnp.concatenate([rel_val, relf_ref[...]], axis=1)
        s8 = lax.dot_general(
            attn_in, wa_ref[...], (((1,), (1,)), ((), ())),
            preferred_element_type=jnp.float32)
        s = s8[:, 0:1]
        s = jnp.where(s >= 0.0, s, 0.2 * s)
        es = jnp.exp(s)
        cnt = jnp.sum(cntp_ref[...], axis=0, keepdims=True)
        denom = lax.dot_general(
            cnt, es, (((1,), (0,)), ((), ())),
            preferred_element_type=jnp.float32) + 1e-9
        v_scr[...] = rel_val * (es / denom)

    x = lax.dot_general(
        c_ref[0:M, :], v_scr[...], (((1,), (0,)), ((), ())),
        preferred_element_type=jnp.float32)
    x = jnp.where(x > 0.0, x, jnp.exp(jnp.minimum(x, 0.0)) - 1.0)
    out_ref[...] = x[None]


_tc_call = pl.pallas_call(
    _tc_body,
    grid=(B,),
    in_specs=[
        pl.BlockSpec((BPAD, R), lambda b: (b, 0)),
        pl.BlockSpec((NW, R), lambda b: (0, 0)),
        pl.BlockSpec((R, D), lambda b: (0, 0)),
        pl.BlockSpec((D, D), lambda b: (0, 0)),
        pl.BlockSpec((8, D), lambda b: (0, 0)),
        pl.BlockSpec((8, 2 * D), lambda b: (0, 0)),
    ],
    out_specs=pl.BlockSpec((1, M, D), lambda b: (b, 0, 0)),
    out_shape=jax.ShapeDtypeStruct((B, M, D), jnp.float32),
    scratch_shapes=[pltpu.VMEM((R, D), jnp.float32)],
)


def kernel(local_entity, batch_heads, batch_rels, batch_tails, batch_ids,
           fact_ids, weight_list, weight_rel_list, rel_features, W_kb, b_kb,
           W_attn):
    heads = batch_heads.astype(jnp.int32)
    tails = batch_tails.astype(jnp.int32)
    ids = batch_ids.astype(jnp.int32)
    rels = batch_rels.astype(jnp.int32)

    hcells, tcells = _prep_call(heads, tails, ids, rels)
    staged, lens, cnt_part = _sc_bin(hcells, tcells, rels)
    zslab = jnp.zeros((SLAB_E, R), jnp.float32)
    c_mat = _sc_hist(staged, lens, zslab)

    bkb8 = jnp.zeros((8, D), jnp.float32).at[0].set(b_kb.astype(jnp.float32))
    wa8 = jnp.zeros((8, 2 * D), jnp.float32).at[0].set(
        W_attn.astype(jnp.float32).reshape(2 * D))

    return _tc_call(c_mat, cnt_part, rel_features, W_kb, bkb8, wa8)


# speedup vs baseline: 1.0902x; 1.0261x over previous
import functools
import jax
import jax.numpy as jnp
from jax import lax
from jax.experimental import pallas as pl
from jax.experimental.pallas import tpu as pltpu
from jax.experimental.pallas import tpu_sc as plsc

B, M, E, R, D = 4, 2500, 160000, 512, 256
BPAD = 2560
PAD_ROWS = B * BPAD
SLAB_E = 160
SLAB_CELLS = SLAB_E * R
NSLAB = PAD_ROWS // SLAB_E
NW = 32
CAP = 256
CELLS_PER_TILE = 2 * E // NW
FACTS_PER_TILE = E // NW


_GATHER_DNUMS = lax.GatherDimensionNumbers(
    offset_dims=(), collapsed_slice_dims=(0,), start_index_map=(0,))


def _take16(v, idx):
    return lax.gather(v, idx[:, None], _GATHER_DNUMS, slice_sizes=(1,),
                      mode=lax.GatherScatterMode.PROMISE_IN_BOUNDS)


def _prep_body(h_ref, t_ref, i_ref, r_ref, hc_ref, tc_ref):
    common = r_ref[...] + i_ref[...] * (BPAD * R)
    hc_ref[...] = h_ref[...] * R + common
    tc_ref[...] = t_ref[...] * R + common


_prep_call = pl.pallas_call(
    _prep_body,
    in_specs=[
        pl.BlockSpec((E,), lambda: (0,)),
        pl.BlockSpec((E,), lambda: (0,)),
        pl.BlockSpec((E,), lambda: (0,)),
        pl.BlockSpec((E,), lambda: (0,)),
    ],
    out_specs=[
        pl.BlockSpec((E,), lambda: (0,)),
        pl.BlockSpec((E,), lambda: (0,)),
    ],
    out_shape=[
        jax.ShapeDtypeStruct((E,), jnp.int32),
        jax.ShapeDtypeStruct((E,), jnp.int32),
    ],
)


_sc_mesh = plsc.VectorSubcoreMesh(core_axis_name="c", subcore_axis_name="s")


@functools.partial(
    pl.kernel,
    out_type=(
        jax.ShapeDtypeStruct((NW * NSLAB, CAP), jnp.int32),
        jax.ShapeDtypeStruct((NW * NSLAB,), jnp.int32),
        jax.ShapeDtypeStruct((NW, R), jnp.float32),
    ),
    mesh=_sc_mesh,
    compiler_params=pltpu.CompilerParams(needs_layout_passes=False),
    scratch_types=(
        pltpu.VMEM((NSLAB, CAP), jnp.int32),
        pltpu.VMEM((CELLS_PER_TILE,), jnp.int32),
        pltpu.VMEM((NSLAB,), jnp.int32),
        pltpu.VMEM((R,), jnp.float32),
    ),
)
def _sc_bin(hcells_hbm, tcells_hbm, rels_hbm, staged_out, lens_out, cnt_out,
            binbuf, cellbuf, cursors, cnthist):
    wid = lax.axis_index("s") * 2 + lax.axis_index("c")
    ones = jnp.ones((16,), jnp.float32)
    iota = lax.iota(jnp.int32, 16)

    fbase = wid * FACTS_PER_TILE
    pltpu.sync_copy(hcells_hbm.at[pl.ds(fbase, FACTS_PER_TILE)],
                    cellbuf.at[pl.ds(0, FACTS_PER_TILE)])
    pltpu.sync_copy(tcells_hbm.at[pl.ds(fbase, FACTS_PER_TILE)],
                    cellbuf.at[pl.ds(FACTS_PER_TILE, FACTS_PER_TILE)])

    for k in range(NSLAB // 16):
        cursors[pl.ds(k * 16, 16)] = jnp.zeros((16,), jnp.int32)

    def bin_body(i, _):
        c = cellbuf[pl.ds(i * 16, 16)]
        row = lax.shift_right_logical(c, 9)
        b = lax.shift_right_logical(row * 6554, 20)
        sk, sv = plsc.sort_key_val(b, c)
        prev = _take16(sk, jnp.maximum(iota - 1, 0))
        boundary = (iota == 0) | (sk != prev)
        firstpos = plsc.cummax(jnp.where(boundary, iota, 0))
        rank = iota - firstpos
        base = plsc.load_gather(cursors, [sk])
        pos = jnp.minimum(base + rank, CAP - 1)
        plsc.store_scatter(binbuf, [sk, pos], sv)
        nxt = _take16(sk, jnp.minimum(iota + 1, 15))
        run_end = (iota == 15) | (sk != nxt)
        plsc.store_scatter(cursors, [sk], pos + 1, mask=run_end)
        return 0
    lax.fori_loop(0, CELLS_PER_TILE // 16, bin_body, 0)

    pltpu.sync_copy(binbuf, staged_out.at[pl.ds(wid * NSLAB, NSLAB)])
    pltpu.sync_copy(cursors, lens_out.at[pl.ds(wid * NSLAB, NSLAB)])

    @plsc.parallel_loop(0, R, step=16, unroll=8)
    def _(i):
        cnthist[pl.ds(i, 16)] = jnp.zeros((16,), jnp.float32)

    pltpu.sync_copy(rels_hbm.at[pl.ds(fbase, FACTS_PER_TILE)],
                    cellbuf.at[pl.ds(0, FACTS_PER_TILE)])

    @plsc.parallel_loop(0, FACTS_PER_TILE, step=16, unroll=8)
    def _(i):
        r = cellbuf[pl.ds(i, 16)]
        m = iota < (FACTS_PER_TILE - i)
        plsc.addupdate_scatter(cnthist, [jnp.where(m, r, 0)], ones, mask=m)

    pltpu.sync_copy(cnthist, cnt_out.at[wid])


@functools.partial(
    pl.kernel,
    out_type=jax.ShapeDtypeStruct((PAD_ROWS, R), jnp.float32),
    mesh=_sc_mesh,
    compiler_params=pltpu.CompilerParams(needs_layout_passes=False),
    scratch_types=(
        pltpu.VMEM((SLAB_E, R), jnp.float32),
        pltpu.VMEM((NW * CAP,), jnp.int32),
        pltpu.VMEM((NW * CAP,), jnp.int32),
        pltpu.VMEM((NW * NSLAB,), jnp.int32),
        pltpu.SemaphoreType.DMA,
        pltpu.SemaphoreType.DMA,
    ),
)
def _sc_hist(staged_hbm, lens_hbm, zslab_hbm, c_out,
             hist, seg0, seg1, lensbuf, sem0, sem1):
    wid = lax.axis_index("s") * 2 + lax.axis_index("c")
    ones = jnp.ones((16,), jnp.float32)
    iota = lax.iota(jnp.int32, 16)
    sb0 = wid * 2
    sb1 = wid * 2 + 1

    pltpu.sync_copy(lens_hbm, lensbuf)
    for s in range(NW):
        pltpu.async_copy(staged_hbm.at[s * NSLAB + sb0],
                         seg0.at[pl.ds(s * CAP, CAP)], sem0)
    pltpu.sync_copy(zslab_hbm, hist)
    for s in range(NW):
        pltpu.make_async_copy(staged_hbm.at[s * NSLAB + sb0],
                              seg0.at[pl.ds(s * CAP, CAP)], sem0).wait()
    for s in range(NW):
        pltpu.async_copy(staged_hbm.at[s * NSLAB + sb1],
                         seg1.at[pl.ds(s * CAP, CAP)], sem1)

    def process_bin(seg, sb):
        base = sb * SLAB_CELLS
        for s in range(NW):
            ls = plsc.load_gather(
                lensbuf, [jnp.full((16,), s * NSLAB + sb, jnp.int32)])

            @plsc.parallel_loop(0, CAP, step=16, unroll=16)
            def _(i):
                m = (iota + i) < ls
                c = seg[pl.ds(s * CAP + i, 16)]
                l = c - base
                lm = jnp.where(m, l, 0)
                row = lax.shift_right_logical(lm, 9)
                col = lm & (R - 1)
                plsc.addupdate_scatter(hist, [row, col], ones, mask=m)

    process_bin(seg0, sb0)
    pltpu.sync_copy(hist, c_out.at[pl.ds(sb0 * SLAB_E, SLAB_E)])
    pltpu.sync_copy(zslab_hbm, hist)

    for s in range(NW):
        pltpu.make_async_copy(staged_hbm.at[s * NSLAB + sb1],
                              seg1.at[pl.ds(s * CAP, CAP)], sem1).wait()
    process_bin(seg1, sb1)
    pltpu.sync_copy(hist, c_out.at[pl.ds(sb1 * SLAB_E, SLAB_E)])


def _tc_body(c_ref, cntp_ref, relf_ref, wkb_ref, bkb_ref, wa_ref, out_ref, v_scr):
    @pl.when(pl.program_id(0) == 0)
    def _():
        rel_val = lax.dot_general(
            relf_ref[...], wkb_ref[...], (((1,), (1,)), ((), ())),
            preferred_element_type=jnp.float32) + bkb_ref[0:1, :]
        attn_in = jnp.concatenate([rel_val, relf_ref[...]], axis=1)
        s8 = lax.dot_general(
            attn_in, wa_ref[...], (((1,), (1,)), ((), ())),
            preferred_element_type=jnp.float32)
        s = s8[:, 0:1]
        s = jnp.where(s >= 0.0, s, 0.2 * s)
        es = jnp.exp(s)
        cnt = jnp.sum(cntp_ref[...], axis=0, keepdims=True)
        denom = lax.dot_general(
            cnt, es, (((1,), (0,)), ((), ())),
            preferred_element_type=jnp.float32) + 1e-9
        v_scr[...] = rel_val * (es / denom)

    x = lax.dot_general(
        c_ref[0:M, :], v_scr[...], (((1,), (0,)), ((), ())),
        preferred_element_type=jnp.float32)
    x = jnp.where(x > 0.0, x, jnp.exp(jnp.minimum(x, 0.0)) - 1.0)
    out_ref[...] = x[None]


_tc_call = pl.pallas_call(
    _tc_body,
    grid=(B,),
    in_specs=[
        pl.BlockSpec((BPAD, R), lambda b: (b, 0)),
        pl.BlockSpec((NW, R), lambda b: (0, 0)),
        pl.BlockSpec((R, D), lambda b: (0, 0)),
        pl.BlockSpec((D, D), lambda b: (0, 0)),
        pl.BlockSpec((8, D), lambda b: (0, 0)),
        pl.BlockSpec((8, 2 * D), lambda b: (0, 0)),
    ],
    out_specs=pl.BlockSpec((1, M, D), lambda b: (b, 0, 0)),
    out_shape=jax.ShapeDtypeStruct((B, M, D), jnp.float32),
    scratch_shapes=[pltpu.VMEM((R, D), jnp.float32)],
)


def kernel(local_entity, batch_heads, batch_rels, batch_tails, batch_ids,
           fact_ids, weight_list, weight_rel_list, rel_features, W_kb, b_kb,
           W_attn):
    heads = batch_heads.astype(jnp.int32)
    tails = batch_tails.astype(jnp.int32)
    ids = batch_ids.astype(jnp.int32)
    rels = batch_rels.astype(jnp.int32)

    hcells, tcells = _prep_call(heads, tails, ids, rels)
    staged, lens, cnt_part = _sc_bin(hcells, tcells, rels)
    zslab = jnp.zeros((SLAB_E, R), jnp.float32)
    c_mat = _sc_hist(staged, lens, zslab)

    bkb8 = jnp.zeros((8, D), jnp.float32).at[0].set(b_kb.astype(jnp.float32))
    wa8 = jnp.zeros((8, 2 * D), jnp.float32).at[0].set(
        W_attn.astype(jnp.float32).reshape(2 * D))

    return _tc_call(c_mat, cnt_part, rel_features, W_kb, bkb8, wa8)
